# baseline (speedup 1.0000x reference)
import functools

import jax
import jax.numpy as jnp
from jax import lax
from jax.experimental import pallas as pl
from jax.experimental.pallas import tpu as pltpu
from jax.experimental.pallas import tpu_sc as plsc

_MAXLEN = 2048
_EMBED = 64
_BATCH = 64
_NW = 32
_ROWS = _BATCH * _MAXLEN
_RPW = _ROWS // _NW
_CHUNK = 512
_NCHUNK = _RPW // _CHUNK
_SUB = 128
_NSUB = _CHUNK // _SUB


@functools.partial(
    pl.kernel,
    mesh=plsc.VectorSubcoreMesh(core_axis_name="c", subcore_axis_name="s"),
    out_type=jax.ShapeDtypeStruct((_ROWS, _EMBED), jnp.float32),
    scratch_types=[
        pltpu.VMEM((_CHUNK,), jnp.int32),
        pltpu.VMEM((_CHUNK, _EMBED), jnp.float32),
        pltpu.SemaphoreType.DMA,
    ],
    compiler_params=pltpu.CompilerParams(use_tc_tiling_on_sc=False),
)
def _gather_rows(xt_hbm, tok_hbm, out_hbm, idx_v, tok_v, gsem):
    wid = lax.axis_index("s") * 2 + lax.axis_index("c")
    base = wid * _RPW

    def chunk_body(c, carry):
        gbase = base + c * _CHUNK
        pltpu.sync_copy(xt_hbm.at[pl.ds(gbase, _CHUNK)], idx_v)
        gathers = [
            pltpu.async_copy(
                tok_hbm.at[idx_v.at[pl.ds(k * _SUB, _SUB)]],
                tok_v.at[pl.ds(k * _SUB, _SUB)],
                gsem,
            )
            for k in range(_NSUB)
        ]
        for cp in gathers:
            cp.wait()
        pltpu.sync_copy(tok_v, out_hbm.at[pl.ds(gbase, _CHUNK)])
        return carry

    lax.fori_loop(0, _NCHUNK, chunk_body, 0)


def kernel(x, token_table, pos_table):
    xt = x.T.reshape(-1).astype(jnp.int32)
    tok = _gather_rows(xt, token_table)
    tok3 = tok.reshape(_MAXLEN, x.shape[0], _EMBED)
    return (tok3 + pos_table[:, None, :]).transpose(1, 0, 2)


# speedup vs baseline: 1.0720x; 1.0720x over previous
import functools

import jax
import jax.numpy as jnp
from jax import lax
from jax.experimental import pallas as pl
from jax.experimental.pallas import tpu as pltpu
from jax.experimental.pallas import tpu_sc as plsc

_MAXLEN = 2048
_EMBED = 64
_BATCH = 64
_NW = 32
_ROWS = _BATCH * _MAXLEN
_RPW = _ROWS // _NW
_PPW = _RPW // _BATCH
_CHUNK = 512
_NCHUNK = _RPW // _CHUNK
_PPC = _CHUNK // _BATCH
_SUB = 128
_NSUB = _CHUNK // _SUB
_LANES = 16


@functools.partial(
    pl.kernel,
    mesh=plsc.VectorSubcoreMesh(core_axis_name="c", subcore_axis_name="s"),
    out_type=jax.ShapeDtypeStruct((_ROWS, _EMBED), jnp.float32),
    scratch_types=[
        pltpu.VMEM((2, _CHUNK), jnp.int32),
        pltpu.VMEM((_CHUNK, _EMBED), jnp.float32),
        pltpu.VMEM((_CHUNK, _EMBED), jnp.float32),
        pltpu.VMEM((_PPW, _EMBED), jnp.float32),
        pltpu.SemaphoreType.DMA,
        pltpu.SemaphoreType.DMA,
        pltpu.SemaphoreType.DMA,
    ],
    compiler_params=pltpu.CompilerParams(use_tc_tiling_on_sc=False),
)
def _tokpos(xt_hbm, tok_hbm, pos_hbm, out_hbm, idx_v, tok_a, tok_b, pos_v,
            gsem_a, gsem_b, wsem):
    wid = lax.axis_index("s") * 2 + lax.axis_index("c")
    base = wid * _RPW
    pltpu.sync_copy(pos_hbm.at[pl.ds(wid * _PPW, _PPW)], pos_v)

    def fire(c, buf, slot, sem):
        gbase = base + c * _CHUNK
        pltpu.sync_copy(xt_hbm.at[pl.ds(gbase, _CHUNK)], idx_v.at[slot])
        return [
            pltpu.async_copy(
                tok_hbm.at[idx_v.at[slot, pl.ds(k * _SUB, _SUB)]],
                buf.at[pl.ds(k * _SUB, _SUB)],
                sem,
            )
            for k in range(_NSUB)
        ]

    def process(c, buf, gathers):
        for cp in gathers:
            cp.wait()
        for q in range(_PPC):
            row0 = q * _BATCH
            pos_regs = [pos_v[c * _PPC + q, pl.ds(e * _LANES, _LANES)]
                        for e in range(_EMBED // _LANES)]

            def body(r, regs):
                for e in range(_EMBED // _LANES):
                    sl = pl.ds(e * _LANES, _LANES)
                    buf[row0 + r, sl] = buf[row0 + r, sl] + regs[e]
                return regs

            lax.fori_loop(0, _BATCH, body, tuple(pos_regs))
        return pltpu.async_copy(
            buf, out_hbm.at[pl.ds(base + c * _CHUNK, _CHUNK)], wsem
        )

    ga = fire(0, tok_a, 0, gsem_a)
    wa = wb = None
    for i in range(_NCHUNK // 2):
        if wb is not None:
            wb.wait()
        gb = fire(2 * i + 1, tok_b, 1, gsem_b)
        wa = process(2 * i, tok_a, ga)
        if 2 * i + 2 < _NCHUNK:
            wa.wait()
            wa = None
            ga = fire(2 * i + 2, tok_a, 0, gsem_a)
        wb = process(2 * i + 1, tok_b, gb)
    if wa is not None:
        wa.wait()
    wb.wait()


def kernel(x, token_table, pos_table):
    xt = x.T.reshape(-1).astype(jnp.int32)
    out_t = _tokpos(xt, token_table, pos_table)
    return out_t.reshape(_MAXLEN, x.shape[0], _EMBED).transpose(1, 0, 2)


# speedup vs baseline: 1.0934x; 1.0199x over previous
import functools

import jax
import jax.numpy as jnp
from jax import lax
from jax.experimental import pallas as pl
from jax.experimental.pallas import tpu as pltpu
from jax.experimental.pallas import tpu_sc as plsc

_MAXLEN = 2048
_EMBED = 64
_BATCH = 64
_NW = 32
_ROWS = _BATCH * _MAXLEN
_RPW = _ROWS // _NW
_PPW = _RPW // _BATCH
_CHUNK = 512
_NCHUNK = _RPW // _CHUNK
_PPC = _CHUNK // _BATCH
_SUB = 128
_NSUB = _CHUNK // _SUB
_LANES = 16


@functools.partial(
    pl.kernel,
    mesh=plsc.VectorSubcoreMesh(core_axis_name="c", subcore_axis_name="s"),
    out_type=jax.ShapeDtypeStruct((_ROWS, _EMBED), jnp.float32),
    scratch_types=[
        pltpu.VMEM((2, _CHUNK), jnp.int32),
        pltpu.VMEM((_CHUNK, _EMBED), jnp.float32),
        pltpu.VMEM((_CHUNK, _EMBED), jnp.float32),
        pltpu.VMEM((_PPW, _EMBED), jnp.float32),
        pltpu.VMEM((2, _NSUB, _SUB), jnp.int32),
        pltpu.SemaphoreType.DMA,
        pltpu.SemaphoreType.DMA,
        pltpu.SemaphoreType.DMA,
    ],
    compiler_params=pltpu.CompilerParams(use_tc_tiling_on_sc=False),
)
def _tokpos(xt_hbm, tok_hbm, pos_hbm, out_hbm, idx_v, tok_a, tok_b, pos_v,
            oidx_v, gsem_a, gsem_b, wsem):
    wid = lax.axis_index("s") * 2 + lax.axis_index("c")
    base = wid * _RPW
    pbase = wid * _PPW
    pltpu.sync_copy(pos_hbm.at[pl.ds(pbase, _PPW)], pos_v)
    iota_b = lax.iota(jnp.int32, _LANES) * _MAXLEN

    def fire(c, buf, slot, sem):
        gbase = base + c * _CHUNK
        pltpu.sync_copy(xt_hbm.at[pl.ds(gbase, _CHUNK)], idx_v.at[slot])
        gathers = [
            pltpu.async_copy(
                tok_hbm.at[idx_v.at[slot, pl.ds(k * _SUB, _SUB)]],
                buf.at[pl.ds(k * _SUB, _SUB)],
                sem,
            )
            for k in range(_NSUB)
        ]
        for k in range(_NSUB):
            for h in range(_SUB // _LANES):
                p_abs = pbase + c * _PPC + 2 * k + h // (_BATCH // _LANES)
                b_off = (h % (_BATCH // _LANES)) * _LANES * _MAXLEN
                oidx_v[slot, k, pl.ds(h * _LANES, _LANES)] = (
                    iota_b + (b_off + p_abs)
                )
        return gathers

    def process(c, buf, slot, gathers):
        for cp in gathers:
            cp.wait()
        for q in range(_PPC):
            row0 = q * _BATCH
            pos_regs = [pos_v[c * _PPC + q, pl.ds(e * _LANES, _LANES)]
                        for e in range(_EMBED // _LANES)]

            def body(r, regs):
                for e in range(_EMBED // _LANES):
                    sl = pl.ds(e * _LANES, _LANES)
                    buf[row0 + r, sl] = buf[row0 + r, sl] + regs[e]
                return regs

            lax.fori_loop(0, _BATCH, body, tuple(pos_regs))
        return [
            pltpu.async_copy(
                buf.at[pl.ds(k * _SUB, _SUB)],
                out_hbm.at[oidx_v.at[slot, k]],
                wsem,
            )
            for k in range(_NSUB)
        ]

    ga = fire(0, tok_a, 0, gsem_a)
    wa = wb = None
    for i in range(_NCHUNK // 2):
        if wb is not None:
            for w in wb:
                w.wait()
        gb = fire(2 * i + 1, tok_b, 1, gsem_b)
        wa = process(2 * i, tok_a, 0, ga)
        if 2 * i + 2 < _NCHUNK:
            for w in wa:
                w.wait()
            wa = None
            ga = fire(2 * i + 2, tok_a, 0, gsem_a)
        wb = process(2 * i + 1, tok_b, 1, gb)
    if wa is not None:
        for w in wa:
            w.wait()
    for w in wb:
        w.wait()


def kernel(x, token_table, pos_table):
    xt = x.T.reshape(-1).astype(jnp.int32)
    out = _tokpos(xt, token_table, pos_table)
    return out.reshape(x.shape[0], x.shape[1], _EMBED)
